# initial kernel scaffold (unmeasured)
import jax
import jax.numpy as jnp
from jax import lax
from jax.experimental import pallas as pl
from jax.experimental.pallas import tpu as pltpu


def kernel(
    x,
):
    def body(*refs):
        pass

    out_shape = jax.ShapeDtypeStruct(..., jnp.float32)
    return pl.pallas_call(body, out_shape=out_shape)(...)


# baseline (device time: 28251 ns/iter reference)
import functools

import jax
import jax.numpy as jnp
from jax import lax
from jax.experimental import pallas as pl
from jax.experimental.pallas import tpu as pltpu

N_STAGES = 4


def kernel(x):
    m, n = x.shape[-2], x.shape[-1]
    x2 = x.reshape(m, n)

    def body(x_ref, out_ref, recv_buf, send_sems, recv_sems):
        mx = lax.axis_index("x")
        my = lax.axis_index("y")
        mz = lax.axis_index("z")

        partners = [
            (mx, my, mz ^ 1),
            (mx, my, mz ^ 2),
            (mx, my ^ 1, mz),
            (mx ^ 1, my, mz),
        ]

        barrier_sem = pltpu.get_barrier_semaphore()
        for p in partners:
            pl.semaphore_signal(
                barrier_sem, inc=1, device_id=p,
                device_id_type=pl.DeviceIdType.MESH,
            )
        pl.semaphore_wait(barrier_sem, N_STAGES)

        out_ref[:, :] = x_ref[:, :]

        for h, p in enumerate(partners):
            rdma = pltpu.make_async_remote_copy(
                src_ref=out_ref,
                dst_ref=recv_buf.at[h],
                send_sem=send_sems.at[h],
                recv_sem=recv_sems.at[h],
                device_id=p,
                device_id_type=pl.DeviceIdType.MESH,
            )
            rdma.start()
            rdma.wait()
            out_ref[:, :] = out_ref[:, :] + recv_buf[h]

        @functools.partial(
            pl.run_scoped, exit_sem=pltpu.SemaphoreType.REGULAR
        )
        def _(exit_sem):
            for p in partners:
                pl.semaphore_signal(
                    exit_sem, inc=1, device_id=p,
                    device_id_type=pl.DeviceIdType.MESH,
                )
            pl.semaphore_wait(exit_sem, N_STAGES)

    return pl.pallas_call(
        body,
        out_shape=jax.ShapeDtypeStruct((m, n), x2.dtype),
        in_specs=[pl.BlockSpec(memory_space=pltpu.VMEM)],
        out_specs=pl.BlockSpec(memory_space=pltpu.VMEM),
        scratch_shapes=[
            pltpu.VMEM((N_STAGES, m, n), x2.dtype),
            pltpu.SemaphoreType.DMA((N_STAGES,)),
            pltpu.SemaphoreType.DMA((N_STAGES,)),
        ],
        compiler_params=pltpu.CompilerParams(collective_id=0),
    )(x2)


# device time: 23646 ns/iter; 1.1947x vs baseline; 1.1947x over previous
import functools

import jax
import jax.numpy as jnp
from jax import lax
from jax.experimental import pallas as pl
from jax.experimental.pallas import tpu as pltpu

N_STAGES = 4


def kernel(x):
    m, n = x.shape[-2], x.shape[-1]
    x2 = x.reshape(m, n)
    hm = m // 2

    def body(x_ref, out_ref, recv_a, recv_b, send_sems, recv_sems):
        mx = lax.axis_index("x")
        my = lax.axis_index("y")
        mz = lax.axis_index("z")

        p_z1 = (mx, my, mz ^ 1)
        p_z2 = (mx, my, mz ^ 2)
        p_y = (mx, my ^ 1, mz)
        p_x = (mx ^ 1, my, mz)

        order_a = [p_z1, p_z2, p_y, p_x]
        order_b = [p_y, p_x, p_z1, p_z2]

        barrier_sem = pltpu.get_barrier_semaphore()
        for p in [p_z1, p_z2, p_y, p_x]:
            pl.semaphore_signal(
                barrier_sem, inc=1, device_id=p,
                device_id_type=pl.DeviceIdType.MESH,
            )
        pl.semaphore_wait(barrier_sem, N_STAGES)

        out_ref[:, :] = x_ref[:, :]

        for h in range(N_STAGES):
            rdma_a = pltpu.make_async_remote_copy(
                src_ref=out_ref.at[pl.ds(0, hm), :],
                dst_ref=recv_a.at[h],
                send_sem=send_sems.at[h],
                recv_sem=recv_sems.at[h],
                device_id=order_a[h],
                device_id_type=pl.DeviceIdType.MESH,
            )
            rdma_b = pltpu.make_async_remote_copy(
                src_ref=out_ref.at[pl.ds(hm, hm), :],
                dst_ref=recv_b.at[h],
                send_sem=send_sems.at[N_STAGES + h],
                recv_sem=recv_sems.at[N_STAGES + h],
                device_id=order_b[h],
                device_id_type=pl.DeviceIdType.MESH,
            )
            rdma_a.start()
            rdma_b.start()
            rdma_a.wait()
            rdma_b.wait()
            out_ref[pl.ds(0, hm), :] = out_ref[pl.ds(0, hm), :] + recv_a[h]
            out_ref[pl.ds(hm, hm), :] = out_ref[pl.ds(hm, hm), :] + recv_b[h]

        @functools.partial(
            pl.run_scoped, exit_sem=pltpu.SemaphoreType.REGULAR
        )
        def _(exit_sem):
            for p in [p_z1, p_z2, p_y, p_x]:
                pl.semaphore_signal(
                    exit_sem, inc=1, device_id=p,
                    device_id_type=pl.DeviceIdType.MESH,
                )
            pl.semaphore_wait(exit_sem, N_STAGES)

    return pl.pallas_call(
        body,
        out_shape=jax.ShapeDtypeStruct((m, n), x2.dtype),
        in_specs=[pl.BlockSpec(memory_space=pltpu.VMEM)],
        out_specs=pl.BlockSpec(memory_space=pltpu.VMEM),
        scratch_shapes=[
            pltpu.VMEM((N_STAGES, hm, n), x2.dtype),
            pltpu.VMEM((N_STAGES, hm, n), x2.dtype),
            pltpu.SemaphoreType.DMA((2 * N_STAGES,)),
            pltpu.SemaphoreType.DMA((2 * N_STAGES,)),
        ],
        compiler_params=pltpu.CompilerParams(collective_id=0),
    )(x2)


# device time: 23642 ns/iter; 1.1949x vs baseline; 1.0002x over previous
import functools

import jax
import jax.numpy as jnp
from jax import lax
from jax.experimental import pallas as pl
from jax.experimental.pallas import tpu as pltpu

N_STAGES = 4


def kernel(x):
    m, n = x.shape[-2], x.shape[-1]
    x2 = x.reshape(m, n)
    hm = m // 2

    def body(x_ref, out_ref, acc, recv, send_sems, recv_sems):
        mx = lax.axis_index("x")
        my = lax.axis_index("y")
        mz = lax.axis_index("z")

        p_z1 = (mx, my, mz ^ 1)
        p_z2 = (mx, my, mz ^ 2)
        p_y = (mx, my ^ 1, mz)
        p_x = (mx ^ 1, my, mz)
        partners = [p_z1, p_z2, p_y, p_x]

        order_a = [p_z1, p_z2, p_y, p_x]
        order_b = [p_y, p_x, p_z1, p_z2]

        bufs = [x_ref, acc.at[0], acc.at[1], acc.at[2], out_ref]

        barrier_sem = pltpu.get_barrier_semaphore()
        for p in partners:
            pl.semaphore_signal(
                barrier_sem, inc=1, device_id=p,
                device_id_type=pl.DeviceIdType.MESH,
            )
        pl.semaphore_wait(barrier_sem, N_STAGES)

        rdmas = []
        for h in range(N_STAGES):
            src = bufs[h]
            rdma_a = pltpu.make_async_remote_copy(
                src_ref=src.at[pl.ds(0, hm), :],
                dst_ref=recv.at[h, pl.ds(0, hm), :],
                send_sem=send_sems.at[h],
                recv_sem=recv_sems.at[h],
                device_id=order_a[h],
                device_id_type=pl.DeviceIdType.MESH,
            )
            rdma_b = pltpu.make_async_remote_copy(
                src_ref=src.at[pl.ds(hm, hm), :],
                dst_ref=recv.at[h, pl.ds(hm, hm), :],
                send_sem=send_sems.at[N_STAGES + h],
                recv_sem=recv_sems.at[N_STAGES + h],
                device_id=order_b[h],
                device_id_type=pl.DeviceIdType.MESH,
            )
            rdma_a.start()
            rdma_b.start()
            rdma_a.wait_recv()
            rdma_b.wait_recv()
            bufs[h + 1][:, :] = src[:, :] + recv[h]
            rdmas.append((rdma_a, rdma_b))

        for rdma_a, rdma_b in rdmas:
            rdma_a.wait_send()
            rdma_b.wait_send()

        @functools.partial(
            pl.run_scoped, exit_sem=pltpu.SemaphoreType.REGULAR
        )
        def _(exit_sem):
            for p in partners:
                pl.semaphore_signal(
                    exit_sem, inc=1, device_id=p,
                    device_id_type=pl.DeviceIdType.MESH,
                )
            pl.semaphore_wait(exit_sem, N_STAGES)

    return pl.pallas_call(
        body,
        out_shape=jax.ShapeDtypeStruct((m, n), x2.dtype),
        in_specs=[pl.BlockSpec(memory_space=pltpu.VMEM)],
        out_specs=pl.BlockSpec(memory_space=pltpu.VMEM),
        scratch_shapes=[
            pltpu.VMEM((N_STAGES - 1, m, n), x2.dtype),
            pltpu.VMEM((N_STAGES, m, n), x2.dtype),
            pltpu.SemaphoreType.DMA((2 * N_STAGES,)),
            pltpu.SemaphoreType.DMA((2 * N_STAGES,)),
        ],
        compiler_params=pltpu.CompilerParams(collective_id=0),
    )(x2)


# device time: 20134 ns/iter; 1.4031x vs baseline; 1.1742x over previous
import functools

import jax
import jax.numpy as jnp
from jax import lax
from jax.experimental import pallas as pl
from jax.experimental.pallas import tpu as pltpu

N_STAGES = 4


def kernel(x):
    m, n = x.shape[-2], x.shape[-1]
    x2 = x.reshape(m, n)
    hm = m // 2

    def body(x_ref, out_ref, acc, recv, send_sems, recv_sems, late_sem):
        mx = lax.axis_index("x")
        my = lax.axis_index("y")
        mz = lax.axis_index("z")

        p_z1 = (mx, my, mz ^ 1)
        p_z2 = (mx, my, mz ^ 2)
        p_y = (mx, my ^ 1, mz)
        p_x = (mx ^ 1, my, mz)

        order_a = [p_z1, p_z2, p_y, p_x]
        order_b = [p_y, p_x, p_z1, p_z2]

        bufs = [x_ref, acc.at[0], acc.at[1], acc.at[2], out_ref]

        barrier_sem = pltpu.get_barrier_semaphore()
        for p in [p_z1, p_y]:
            pl.semaphore_signal(
                barrier_sem, inc=1, device_id=p,
                device_id_type=pl.DeviceIdType.MESH,
            )
        for p in [p_z2, p_x]:
            pl.semaphore_signal(
                late_sem, inc=1, device_id=p,
                device_id_type=pl.DeviceIdType.MESH,
            )
        pl.semaphore_wait(barrier_sem, 2)

        rdmas = []
        for h in range(N_STAGES):
            if h == 1:
                pl.semaphore_wait(late_sem, 2)
            src = bufs[h]
            rdma_a = pltpu.make_async_remote_copy(
                src_ref=src.at[pl.ds(0, hm), :],
                dst_ref=recv.at[h, pl.ds(0, hm), :],
                send_sem=send_sems.at[h],
                recv_sem=recv_sems.at[h],
                device_id=order_a[h],
                device_id_type=pl.DeviceIdType.MESH,
            )
            rdma_b = pltpu.make_async_remote_copy(
                src_ref=src.at[pl.ds(hm, hm), :],
                dst_ref=recv.at[h, pl.ds(hm, hm), :],
                send_sem=send_sems.at[N_STAGES + h],
                recv_sem=recv_sems.at[N_STAGES + h],
                device_id=order_b[h],
                device_id_type=pl.DeviceIdType.MESH,
            )
            rdma_a.start()
            rdma_b.start()
            rdma_a.wait_recv()
            rdma_b.wait_recv()
            bufs[h + 1][:, :] = src[:, :] + recv[h]
            rdmas.append((rdma_a, rdma_b))

        for rdma_a, rdma_b in rdmas:
            rdma_a.wait_send()
            rdma_b.wait_send()

    return pl.pallas_call(
        body,
        out_shape=jax.ShapeDtypeStruct((m, n), x2.dtype),
        in_specs=[pl.BlockSpec(memory_space=pltpu.VMEM)],
        out_specs=pl.BlockSpec(memory_space=pltpu.VMEM),
        scratch_shapes=[
            pltpu.VMEM((N_STAGES - 1, m, n), x2.dtype),
            pltpu.VMEM((N_STAGES, m, n), x2.dtype),
            pltpu.SemaphoreType.DMA((2 * N_STAGES,)),
            pltpu.SemaphoreType.DMA((2 * N_STAGES,)),
            pltpu.SemaphoreType.REGULAR,
        ],
        compiler_params=pltpu.CompilerParams(collective_id=0),
    )(x2)


# device time: 18000 ns/iter; 1.5695x vs baseline; 1.1186x over previous
import jax
import jax.numpy as jnp
from jax import lax
from jax.experimental import pallas as pl
from jax.experimental.pallas import tpu as pltpu

N_STEPS = 4
N_STREAMS = 4


def kernel(x):
    m, n = x.shape[-2], x.shape[-1]
    x2 = x.reshape(m, n)
    qm = m // N_STREAMS

    def body(x_ref, out_ref, acc, recv, send_sems, recv_sems, late_sem):
        mx = lax.axis_index("x")
        my = lax.axis_index("y")
        mz = lax.axis_index("z")

        p_z1 = (mx, my, mz ^ 1)
        p_z2 = (mx, my, mz ^ 2)
        p_y = (mx, my ^ 1, mz)
        p_x = (mx ^ 1, my, mz)

        order_top = [p_z2, p_z1, p_y, p_x]
        order_bot = [p_y, p_x, p_z2, p_z1]
        top = list(range(N_STREAMS // 2))
        bot = list(range(N_STREAMS // 2, N_STREAMS))
        orders = [order_top] * len(top) + [order_bot] * len(bot)

        bufs = []
        for s in range(N_STREAMS):
            rows = pl.ds(s * qm, qm)
            bufs.append(
                [x_ref.at[rows, :]]
                + [acc.at[3 * s + i] for i in range(3)]
                + [out_ref.at[rows, :]]
            )

        rdmas = {}

        def start(s, t):
            slot = N_STEPS * s + t
            r = pltpu.make_async_remote_copy(
                src_ref=bufs[s][t],
                dst_ref=recv.at[slot],
                send_sem=send_sems.at[slot],
                recv_sem=recv_sems.at[slot],
                device_id=orders[s][t],
                device_id_type=pl.DeviceIdType.MESH,
            )
            r.start()
            rdmas[(s, t)] = r

        def finish(s, t):
            rdmas[(s, t)].wait_recv()
            bufs[s][t + 1][:, :] = bufs[s][t][:, :] + recv[N_STEPS * s + t]
            if t + 1 < N_STEPS:
                start(s, t + 1)

        barrier_sem = pltpu.get_barrier_semaphore()
        for p in [p_z2, p_y]:
            pl.semaphore_signal(
                barrier_sem, inc=1, device_id=p,
                device_id_type=pl.DeviceIdType.MESH,
            )
        for p in [p_z1, p_x]:
            pl.semaphore_signal(
                late_sem, inc=1, device_id=p,
                device_id_type=pl.DeviceIdType.MESH,
            )
        pl.semaphore_wait(barrier_sem, 2)

        for s in bot + top:
            start(s, 0)

        pl.semaphore_wait(late_sem, 2)

        schedule = (
            [(s, 0) for s in bot + top]
            + [(s, 1) for s in bot + top]
            + [(s, 2) for s in top + bot]
            + [(s, 3) for s in top + bot]
        )
        for s, t in schedule:
            finish(s, t)

        for r in rdmas.values():
            r.wait_send()

    return pl.pallas_call(
        body,
        out_shape=jax.ShapeDtypeStruct((m, n), x2.dtype),
        in_specs=[pl.BlockSpec(memory_space=pltpu.VMEM)],
        out_specs=pl.BlockSpec(memory_space=pltpu.VMEM),
        scratch_shapes=[
            pltpu.VMEM((3 * N_STREAMS, qm, n), x2.dtype),
            pltpu.VMEM((N_STEPS * N_STREAMS, qm, n), x2.dtype),
            pltpu.SemaphoreType.DMA((N_STEPS * N_STREAMS,)),
            pltpu.SemaphoreType.DMA((N_STEPS * N_STREAMS,)),
            pltpu.SemaphoreType.REGULAR,
        ],
        compiler_params=pltpu.CompilerParams(collective_id=0),
    )(x2)
